# baseline (speedup 1.0000x reference)
import functools

import jax
import jax.numpy as jnp
from jax import lax
from jax.experimental import pallas as pl
from jax.experimental.pallas import tpu as pltpu
from jax.experimental.pallas import tpu_sc as plsc

N = 100000
EMB = 128
HID = 64
ROWS_BLK = 1792
GRID = 56
NBUF = 6

NC = 2
NS = 16
NW = NC * NS
CHUNK = 3136
NPAD = NW * CHUNK
LANES = 16
TAIL = NPAD - N
VPAD = 100096


def _mlp_body(x_hbm, w1t_ref, b1_ref, w2t_ref, b2_ref, out_ref, xbuf, sems):
    i = pl.program_id(0)

    def start(blk):
        pltpu.make_async_copy(
            x_hbm.at[pl.ds(blk * ROWS_BLK, ROWS_BLK)],
            xbuf.at[blk % NBUF],
            sems.at[blk % NBUF],
        ).start()

    @pl.when(i == 0)
    def _prime():
        for b in range(NBUF):
            start(b)

    pltpu.make_async_copy(
        x_hbm.at[pl.ds(i * ROWS_BLK, ROWS_BLK)],
        xbuf.at[i % NBUF],
        sems.at[i % NBUF],
    ).wait()

    @pl.when(i + NBUF < GRID)
    def _prefetch():
        start(i + NBUF)

    xT = jnp.transpose(xbuf[i % NBUF])
    h = jnp.maximum(
        jnp.dot(w1t_ref[...], xT, preferred_element_type=jnp.float32)
        + b1_ref[...],
        0.0,
    )
    f = jnp.dot(w2t_ref[...], h, preferred_element_type=jnp.float32) + b2_ref[0]
    cols = i * ROWS_BLK + lax.broadcasted_iota(jnp.int32, (1, ROWS_BLK), 1)
    out_ref[...] = jnp.where(cols < N, f, 0.0)[None]


def _mlp_flows(node_emb, W1, b1, W2, b2):
    return pl.pallas_call(
        _mlp_body,
        grid=(GRID,),
        in_specs=[
            pl.BlockSpec(memory_space=pltpu.HBM),
            pl.BlockSpec((HID, EMB), lambda i: (0, 0)),
            pl.BlockSpec((HID, 1), lambda i: (0, 0)),
            pl.BlockSpec((1, HID), lambda i: (0, 0)),
            pl.BlockSpec(memory_space=pltpu.SMEM),
        ],
        out_specs=pl.BlockSpec((1, 1, ROWS_BLK), lambda i: (i, 0, 0)),
        out_shape=jax.ShapeDtypeStruct((GRID, 1, ROWS_BLK), jnp.float32),
        scratch_shapes=[
            pltpu.VMEM((NBUF, ROWS_BLK, EMB), jnp.float32),
            pltpu.SemaphoreType.DMA((NBUF,)),
        ],
    )(node_emb, W1.T, b1.reshape(HID, 1), W2.T, b2)


_SC_MESH = plsc.VectorSubcoreMesh(core_axis_name="c", subcore_axis_name="s")


@functools.partial(
    pl.kernel,
    mesh=_SC_MESH,
    compiler_params=pltpu.CompilerParams(
        use_tc_tiling_on_sc=False, needs_layout_passes=False
    ),
    out_type=(
        jax.ShapeDtypeStruct((NPAD,), jnp.float32),
        jax.ShapeDtypeStruct((NPAD,), jnp.float32),
    ),
    scratch_types=[
        pltpu.VMEM((VPAD,), jnp.float32),
        pltpu.VMEM((CHUNK,), jnp.int32),
        pltpu.VMEM((CHUNK,), jnp.int32),
        pltpu.VMEM((CHUNK,), jnp.float32),
        pltpu.VMEM((CHUNK,), jnp.float32),
    ],
)
def _edges_sc(ei_hbm, z0s_hbm, volt_hbm, vd_hbm, z0_hbm,
              voltv, rowv, colv, vdv, z0v):
    sid = lax.axis_index("s")
    wid = sid * NC + lax.axis_index("c")
    base = wid * CHUNK

    pltpu.sync_copy(volt_hbm, voltv.at[pl.ds(0, N)])
    pltpu.sync_copy(ei_hbm.at[0, pl.ds(base, CHUNK)], rowv)
    pltpu.sync_copy(ei_hbm.at[1, pl.ds(base, CHUNK)], colv)
    pltpu.sync_copy(z0s_hbm.at[pl.ds(base, CHUNK)], z0v)

    nmax = jnp.full((LANES,), N - 1, jnp.int32)
    zero = jnp.zeros((LANES,), jnp.int32)

    def body(i, carry):
        sl = pl.ds(i * LANES, LANES)
        ri = jnp.minimum(jnp.maximum(rowv[sl], zero), nmax)
        ci = jnp.minimum(jnp.maximum(colv[sl], zero), nmax)
        vr = plsc.load_gather(voltv, [ri])
        vc = plsc.load_gather(voltv, [ci])
        vdv[sl] = vr - vc
        return carry

    lax.fori_loop(0, CHUNK // LANES, body, 0)
    pltpu.sync_copy(vdv, vd_hbm.at[pl.ds(base, CHUNK)])
    pltpu.sync_copy(z0v, z0_hbm.at[pl.ds(base, CHUNK)])


RES_GRID = 2
RES_BLK = NPAD // RES_GRID


def _res_body(vd_ref, fl_ref, z0_ref, o_ref):
    i = pl.program_id(0)

    @pl.when(i == 0)
    def _init():
        o_ref[0] = 0.0

    cols = i * RES_BLK + lax.broadcasted_iota(jnp.int32, (1, RES_BLK), 1)
    r = vd_ref[0] - z0_ref[0] * fl_ref[0]
    part = jnp.sum(jnp.where(cols < N, r * r, 0.0))
    o_ref[0] += part

    @pl.when(i == RES_GRID - 1)
    def _fini():
        o_ref[0] = o_ref[0] * (1.0 / N)


def _residual_loss(vd3, fl3, z03):
    return pl.pallas_call(
        _res_body,
        grid=(RES_GRID,),
        in_specs=[
            pl.BlockSpec((1, 1, RES_BLK), lambda i: (i, 0, 0)),
            pl.BlockSpec((1, 1, RES_BLK), lambda i: (i, 0, 0)),
            pl.BlockSpec((1, 1, RES_BLK), lambda i: (i, 0, 0)),
        ],
        out_specs=pl.BlockSpec(memory_space=pltpu.SMEM),
        out_shape=jax.ShapeDtypeStruct((1,), jnp.float32),
    )(vd3, fl3, z03)


def kernel(node_emb, voltages, edge_index, edge_attr, W1, b1, W2, b2):
    ei = edge_index.astype(jnp.int32)
    z0s = edge_attr[:, 0]
    vdiff, z0p = _edges_sc(ei, z0s, voltages)
    flows2 = _mlp_flows(node_emb, W1, b1, W2, b2)
    flows = flows2.reshape(NPAD)[:N]
    vd3 = vdiff.reshape(RES_GRID, 1, RES_BLK)
    z03 = z0p.reshape(RES_GRID, 1, RES_BLK)
    fl3 = flows2.reshape(RES_GRID, 1, RES_BLK)
    loss = _residual_loss(vd3, fl3, z03)[0]
    return (flows, loss)


# speedup vs baseline: 1.2058x; 1.2058x over previous
import functools

import jax
import jax.numpy as jnp
from jax import lax
from jax.experimental import pallas as pl
from jax.experimental.pallas import tpu as pltpu
from jax.experimental.pallas import tpu_sc as plsc

N = 100000
EMB = 128
HID = 64
ROWS_BLK = 7168
GRID = 14
NBUF = 3

NC = 2
NS = 16
NW = NC * NS
CHUNK = 3136
NPAD = NW * CHUNK
LANES = 16
TAIL = NPAD - N
VPAD = 100096


def _mlp_body(x_hbm, w1t_ref, b1_ref, w2t_ref, b2_ref, out_ref, xbuf, sems):
    i = pl.program_id(0)

    def start(blk):
        pltpu.make_async_copy(
            x_hbm.at[pl.ds(blk * ROWS_BLK, ROWS_BLK)],
            xbuf.at[blk % NBUF],
            sems.at[blk % NBUF],
        ).start()

    @pl.when(i == 0)
    def _prime():
        for b in range(NBUF):
            start(b)

    pltpu.make_async_copy(
        x_hbm.at[pl.ds(i * ROWS_BLK, ROWS_BLK)],
        xbuf.at[i % NBUF],
        sems.at[i % NBUF],
    ).wait()

    @pl.when(i + NBUF < GRID)
    def _prefetch():
        start(i + NBUF)

    xT = jnp.transpose(xbuf[i % NBUF])
    h = jnp.maximum(
        jnp.dot(w1t_ref[...], xT, preferred_element_type=jnp.float32)
        + b1_ref[...],
        0.0,
    )
    f = jnp.dot(w2t_ref[...], h, preferred_element_type=jnp.float32) + b2_ref[0]
    cols = i * ROWS_BLK + lax.broadcasted_iota(jnp.int32, (1, ROWS_BLK), 1)
    out_ref[...] = jnp.where(cols < N, f, 0.0)[None]


def _mlp_flows(node_emb, W1, b1, W2, b2):
    return pl.pallas_call(
        _mlp_body,
        grid=(GRID,),
        in_specs=[
            pl.BlockSpec(memory_space=pltpu.HBM),
            pl.BlockSpec((HID, EMB), lambda i: (0, 0)),
            pl.BlockSpec((HID, 1), lambda i: (0, 0)),
            pl.BlockSpec((1, HID), lambda i: (0, 0)),
            pl.BlockSpec(memory_space=pltpu.SMEM),
        ],
        out_specs=pl.BlockSpec((1, 1, ROWS_BLK), lambda i: (i, 0, 0)),
        out_shape=jax.ShapeDtypeStruct((GRID, 1, ROWS_BLK), jnp.float32),
        scratch_shapes=[
            pltpu.VMEM((NBUF, ROWS_BLK, EMB), jnp.float32),
            pltpu.SemaphoreType.DMA((NBUF,)),
        ],
    )(node_emb, W1.T, b1.reshape(HID, 1), W2.T, b2)


_SC_MESH = plsc.VectorSubcoreMesh(core_axis_name="c", subcore_axis_name="s")


@functools.partial(
    pl.kernel,
    mesh=_SC_MESH,
    compiler_params=pltpu.CompilerParams(
        use_tc_tiling_on_sc=False, needs_layout_passes=False
    ),
    out_type=(
        jax.ShapeDtypeStruct((NPAD,), jnp.float32),
        jax.ShapeDtypeStruct((NPAD,), jnp.float32),
    ),
    scratch_types=[
        pltpu.VMEM((VPAD,), jnp.float32),
        pltpu.VMEM((CHUNK,), jnp.int32),
        pltpu.VMEM((CHUNK,), jnp.int32),
        pltpu.VMEM((CHUNK,), jnp.float32),
        pltpu.VMEM((CHUNK,), jnp.float32),
    ],
)
def _edges_sc(ei_hbm, z0s_hbm, volt_hbm, vd_hbm, z0_hbm,
              voltv, rowv, colv, vdv, z0v):
    sid = lax.axis_index("s")
    wid = sid * NC + lax.axis_index("c")
    base = wid * CHUNK

    pltpu.sync_copy(volt_hbm, voltv.at[pl.ds(0, N)])
    pltpu.sync_copy(ei_hbm.at[0, pl.ds(base, CHUNK)], rowv)
    pltpu.sync_copy(ei_hbm.at[1, pl.ds(base, CHUNK)], colv)
    pltpu.sync_copy(z0s_hbm.at[pl.ds(base, CHUNK)], z0v)

    nmax = jnp.full((LANES,), N - 1, jnp.int32)
    zero = jnp.zeros((LANES,), jnp.int32)

    def body(i, carry):
        sl = pl.ds(i * LANES, LANES)
        ri = jnp.minimum(jnp.maximum(rowv[sl], zero), nmax)
        ci = jnp.minimum(jnp.maximum(colv[sl], zero), nmax)
        vr = plsc.load_gather(voltv, [ri])
        vc = plsc.load_gather(voltv, [ci])
        vdv[sl] = vr - vc
        return carry

    lax.fori_loop(0, CHUNK // LANES, body, 0)
    pltpu.sync_copy(vdv, vd_hbm.at[pl.ds(base, CHUNK)])
    pltpu.sync_copy(z0v, z0_hbm.at[pl.ds(base, CHUNK)])


RES_GRID = 2
RES_BLK = NPAD // RES_GRID


def _res_body(vd_ref, fl_ref, z0_ref, o_ref):
    i = pl.program_id(0)

    @pl.when(i == 0)
    def _init():
        o_ref[0] = 0.0

    cols = i * RES_BLK + lax.broadcasted_iota(jnp.int32, (1, RES_BLK), 1)
    r = vd_ref[0] - z0_ref[0] * fl_ref[0]
    part = jnp.sum(jnp.where(cols < N, r * r, 0.0))
    o_ref[0] += part

    @pl.when(i == RES_GRID - 1)
    def _fini():
        o_ref[0] = o_ref[0] * (1.0 / N)


def _residual_loss(vd3, fl3, z03):
    return pl.pallas_call(
        _res_body,
        grid=(RES_GRID,),
        in_specs=[
            pl.BlockSpec((1, 1, RES_BLK), lambda i: (i, 0, 0)),
            pl.BlockSpec((1, 1, RES_BLK), lambda i: (i, 0, 0)),
            pl.BlockSpec((1, 1, RES_BLK), lambda i: (i, 0, 0)),
        ],
        out_specs=pl.BlockSpec(memory_space=pltpu.SMEM),
        out_shape=jax.ShapeDtypeStruct((1,), jnp.float32),
    )(vd3, fl3, z03)


def kernel(node_emb, voltages, edge_index, edge_attr, W1, b1, W2, b2):
    ei = edge_index.astype(jnp.int32)
    z0s = edge_attr[:, 0]
    vdiff, z0p = _edges_sc(ei, z0s, voltages)
    flows2 = _mlp_flows(node_emb, W1, b1, W2, b2)
    flows = flows2.reshape(NPAD)[:N]
    vd3 = vdiff.reshape(RES_GRID, 1, RES_BLK)
    z03 = z0p.reshape(RES_GRID, 1, RES_BLK)
    fl3 = flows2.reshape(RES_GRID, 1, RES_BLK)
    loss = _residual_loss(vd3, fl3, z03)[0]
    return (flows, loss)


# speedup vs baseline: 1.2121x; 1.0052x over previous
import functools

import jax
import jax.numpy as jnp
from jax import lax
from jax.experimental import pallas as pl
from jax.experimental.pallas import tpu as pltpu
from jax.experimental.pallas import tpu_sc as plsc

N = 100000
EMB = 128
HID = 64
ROWS_BLK = 7168
GRID = 14
NBUF = 4

NC = 2
NS = 16
NW = NC * NS
CHUNK = 3136
NPAD = NW * CHUNK
LANES = 16
TAIL = NPAD - N
VPAD = 100096


def _mlp_body(x_hbm, w1t_ref, b1_ref, w2t_ref, b2_ref, out_ref, xbuf, sems):
    i = pl.program_id(0)

    def start(blk):
        pltpu.make_async_copy(
            x_hbm.at[pl.ds(blk * ROWS_BLK, ROWS_BLK)],
            xbuf.at[blk % NBUF],
            sems.at[blk % NBUF],
        ).start()

    @pl.when(i == 0)
    def _prime():
        for b in range(NBUF):
            start(b)

    pltpu.make_async_copy(
        x_hbm.at[pl.ds(i * ROWS_BLK, ROWS_BLK)],
        xbuf.at[i % NBUF],
        sems.at[i % NBUF],
    ).wait()

    @pl.when(i + NBUF < GRID)
    def _prefetch():
        start(i + NBUF)

    xT = jnp.transpose(xbuf[i % NBUF])
    h = jnp.maximum(
        jnp.dot(w1t_ref[...], xT, preferred_element_type=jnp.float32)
        + b1_ref[...],
        0.0,
    )
    f = jnp.dot(w2t_ref[...], h, preferred_element_type=jnp.float32) + b2_ref[0]
    cols = i * ROWS_BLK + lax.broadcasted_iota(jnp.int32, (1, ROWS_BLK), 1)
    out_ref[...] = jnp.where(cols < N, f, 0.0)[None]


def _mlp_flows(node_emb, W1, b1, W2, b2):
    return pl.pallas_call(
        _mlp_body,
        grid=(GRID,),
        in_specs=[
            pl.BlockSpec(memory_space=pltpu.HBM),
            pl.BlockSpec((HID, EMB), lambda i: (0, 0)),
            pl.BlockSpec((HID, 1), lambda i: (0, 0)),
            pl.BlockSpec((1, HID), lambda i: (0, 0)),
            pl.BlockSpec(memory_space=pltpu.SMEM),
        ],
        out_specs=pl.BlockSpec((1, 1, ROWS_BLK), lambda i: (i, 0, 0)),
        out_shape=jax.ShapeDtypeStruct((GRID, 1, ROWS_BLK), jnp.float32),
        scratch_shapes=[
            pltpu.VMEM((NBUF, ROWS_BLK, EMB), jnp.float32),
            pltpu.SemaphoreType.DMA((NBUF,)),
        ],
    )(node_emb, W1.T, b1.reshape(HID, 1), W2.T, b2)


_SC_MESH = plsc.VectorSubcoreMesh(core_axis_name="c", subcore_axis_name="s")


@functools.partial(
    pl.kernel,
    mesh=_SC_MESH,
    compiler_params=pltpu.CompilerParams(
        use_tc_tiling_on_sc=False, needs_layout_passes=False
    ),
    out_type=(
        jax.ShapeDtypeStruct((NPAD,), jnp.float32),
        jax.ShapeDtypeStruct((NPAD,), jnp.float32),
    ),
    scratch_types=[
        pltpu.VMEM((VPAD,), jnp.float32),
        pltpu.VMEM((CHUNK,), jnp.int32),
        pltpu.VMEM((CHUNK,), jnp.int32),
        pltpu.VMEM((CHUNK,), jnp.float32),
        pltpu.VMEM((CHUNK,), jnp.float32),
    ],
)
def _edges_sc(ei_hbm, z0s_hbm, volt_hbm, vd_hbm, z0_hbm,
              voltv, rowv, colv, vdv, z0v):
    sid = lax.axis_index("s")
    wid = sid * NC + lax.axis_index("c")
    base = wid * CHUNK

    pltpu.sync_copy(volt_hbm, voltv.at[pl.ds(0, N)])
    pltpu.sync_copy(ei_hbm.at[0, pl.ds(base, CHUNK)], rowv)
    pltpu.sync_copy(ei_hbm.at[1, pl.ds(base, CHUNK)], colv)
    pltpu.sync_copy(z0s_hbm.at[pl.ds(base, CHUNK)], z0v)

    nmax = jnp.full((LANES,), N - 1, jnp.int32)
    zero = jnp.zeros((LANES,), jnp.int32)

    def body(i, carry):
        sl = pl.ds(i * LANES, LANES)
        ri = jnp.minimum(jnp.maximum(rowv[sl], zero), nmax)
        ci = jnp.minimum(jnp.maximum(colv[sl], zero), nmax)
        vr = plsc.load_gather(voltv, [ri])
        vc = plsc.load_gather(voltv, [ci])
        vdv[sl] = vr - vc
        return carry

    lax.fori_loop(0, CHUNK // LANES, body, 0)
    pltpu.sync_copy(vdv, vd_hbm.at[pl.ds(base, CHUNK)])
    pltpu.sync_copy(z0v, z0_hbm.at[pl.ds(base, CHUNK)])


RES_GRID = 2
RES_BLK = NPAD // RES_GRID


def _res_body(vd_ref, fl_ref, z0_ref, o_ref):
    i = pl.program_id(0)

    @pl.when(i == 0)
    def _init():
        o_ref[0] = 0.0

    cols = i * RES_BLK + lax.broadcasted_iota(jnp.int32, (1, RES_BLK), 1)
    r = vd_ref[0] - z0_ref[0] * fl_ref[0]
    part = jnp.sum(jnp.where(cols < N, r * r, 0.0))
    o_ref[0] += part

    @pl.when(i == RES_GRID - 1)
    def _fini():
        o_ref[0] = o_ref[0] * (1.0 / N)


def _residual_loss(vd3, fl3, z03):
    return pl.pallas_call(
        _res_body,
        grid=(RES_GRID,),
        in_specs=[
            pl.BlockSpec((1, 1, RES_BLK), lambda i: (i, 0, 0)),
            pl.BlockSpec((1, 1, RES_BLK), lambda i: (i, 0, 0)),
            pl.BlockSpec((1, 1, RES_BLK), lambda i: (i, 0, 0)),
        ],
        out_specs=pl.BlockSpec(memory_space=pltpu.SMEM),
        out_shape=jax.ShapeDtypeStruct((1,), jnp.float32),
    )(vd3, fl3, z03)


def kernel(node_emb, voltages, edge_index, edge_attr, W1, b1, W2, b2):
    ei = edge_index.astype(jnp.int32)
    z0s = edge_attr[:, 0]
    vdiff, z0p = _edges_sc(ei, z0s, voltages)
    flows2 = _mlp_flows(node_emb, W1, b1, W2, b2)
    flows = flows2.reshape(NPAD)[:N]
    vd3 = vdiff.reshape(RES_GRID, 1, RES_BLK)
    z03 = z0p.reshape(RES_GRID, 1, RES_BLK)
    fl3 = flows2.reshape(RES_GRID, 1, RES_BLK)
    loss = _residual_loss(vd3, fl3, z03)[0]
    return (flows, loss)
